# baseline (speedup 1.0000x reference)
import functools

import jax
import jax.numpy as jnp
from jax import lax
from jax.experimental import pallas as pl
from jax.experimental.pallas import tpu as pltpu
from jax.experimental.pallas import tpu_sc as plsc

N = 4096
D = 4096
LANES = 16
NC = 1
NS = 16
NW = NC * NS
RPW = N // NW
CH = 8
NCHUNK = RPW // CH
NBUF = 3
UNROLL = 8

_mesh = plsc.VectorSubcoreMesh(core_axis_name="c", subcore_axis_name="s", num_cores=1)


@functools.partial(
    pl.kernel,
    out_type=jax.ShapeDtypeStruct((N, D), jnp.float32),
    mesh=_mesh,
    scratch_types=[
        pltpu.VMEM((RPW,), jnp.int32),
        pltpu.VMEM((RPW * LANES,), jnp.float32),
        pltpu.VMEM((NCHUNK,), jnp.int32),
        pltpu.VMEM((NBUF, CH, D), jnp.float32),
        pltpu.SemaphoreType.DMA,
        pltpu.SemaphoreType.DMA,
        pltpu.SemaphoreType.DMA,
        pltpu.SemaphoreType.DMA,
        pltpu.SemaphoreType.DMA,
        pltpu.SemaphoreType.DMA,
    ],
)
def _rotate_gather(x_hbm, idx_hbm, scale_hbm, flags_hbm, out_hbm,
                   idx_v, scale_v, flags_v, bufs,
                   gs0, gs1, gs2, os0, os1, os2):
    gsems = (gs0, gs1, gs2)
    osems = (os0, os1, os2)
    wid = lax.axis_index("s") * NC + lax.axis_index("c")
    base = wid * RPW

    pltpu.sync_copy(idx_hbm.at[pl.ds(base, RPW)], idx_v)
    pltpu.sync_copy(scale_hbm.at[pl.ds(base * LANES, RPW * LANES)], scale_v)
    pltpu.sync_copy(flags_hbm.at[pl.ds(wid * NCHUNK, NCHUNK)], flags_v)

    def gather(g):
        b = g % NBUF
        return pltpu.async_copy(
            x_hbm.at[idx_v.at[pl.ds(g * CH, CH)]], bufs.at[b], gsems[b])

    def scatter(g):
        b = g % NBUF
        return pltpu.async_copy(
            bufs.at[b], out_hbm.at[pl.ds(base + g * CH, CH)], osems[b])

    def compute(g):
        b = g % NBUF
        fvec = flags_v[pl.ds((g // LANES) * LANES, LANES)]
        flag = fvec[g % LANES]

        @pl.when(flag != 0)
        def _():
            def row_body(r, carry):
                svec = scale_v[pl.ds((g * CH + r) * LANES, LANES)]

                def col_body(j, carry2):
                    off = j * (LANES * UNROLL)
                    for u in range(UNROLL):
                        sl = pl.ds(off + u * LANES, LANES)
                        bufs[b, r, sl] = bufs[b, r, sl] * svec
                    return carry2

                return lax.fori_loop(0, D // (LANES * UNROLL), col_body, carry)

            lax.fori_loop(0, CH, row_body, 0)

    gc = {}
    oc = {}
    for g in range(min(NBUF, NCHUNK)):
        gc[g] = gather(g)
    for g in range(NCHUNK):
        gc[g].wait()
        compute(g)
        oc[g] = scatter(g)
        ng = g + NBUF - 1
        if NBUF <= ng < NCHUNK:
            oc[ng - NBUF].wait()
            gc[ng] = gather(ng)
    for g in range(max(0, NCHUNK - NBUF), NCHUNK):
        oc[g].wait()


def kernel(inputs):
    n = inputs.shape[0]
    key = jax.random.key(42)
    k_exec, k_flip, k_perm = jax.random.split(key, 3)
    execute = jax.random.uniform(k_exec, (), minval=0.0, maxval=1.0) < 0.1
    flip = jax.random.randint(k_flip, (n,), -1, 1).astype(jnp.float32)
    rotate_axis = jax.random.permutation(k_perm, n)
    idx = jnp.where(execute, rotate_axis,
                    jnp.arange(n, dtype=rotate_axis.dtype)).astype(jnp.int32)
    scale = jnp.where(execute, flip, jnp.ones((n,), jnp.float32))
    scale_exp = jnp.repeat(scale, LANES)
    flags = jnp.any(
        scale.reshape(n // CH, CH) != 1.0, axis=1).astype(jnp.int32)
    return _rotate_gather(inputs, idx, scale_exp, flags)


# speedup vs baseline: 1.1854x; 1.1854x over previous
import functools

import jax
import jax.numpy as jnp
from jax import lax
from jax.experimental import pallas as pl
from jax.experimental.pallas import tpu as pltpu
from jax.experimental.pallas import tpu_sc as plsc

N = 4096
D = 4096
LANES = 16
NC = 2
NS = 16
NW = NC * NS
RPW = N // NW
CH = 8
NCHUNK = RPW // CH
NBUF = 3
UNROLL = 8

_mesh = plsc.VectorSubcoreMesh(core_axis_name="c", subcore_axis_name="s")


@functools.partial(
    pl.kernel,
    out_type=jax.ShapeDtypeStruct((N, D), jnp.float32),
    mesh=_mesh,
    scratch_types=[
        pltpu.VMEM((RPW,), jnp.int32),
        pltpu.VMEM((RPW * LANES,), jnp.float32),
        pltpu.VMEM((NCHUNK,), jnp.int32),
        pltpu.VMEM((NBUF, CH, D), jnp.float32),
        pltpu.SemaphoreType.DMA,
        pltpu.SemaphoreType.DMA,
        pltpu.SemaphoreType.DMA,
        pltpu.SemaphoreType.DMA,
        pltpu.SemaphoreType.DMA,
        pltpu.SemaphoreType.DMA,
        pltpu.SemaphoreType.DMA,
    ],
)
def _rotate_gather(x_hbm, idx_hbm, scale_hbm, flags_hbm, out_hbm,
                   idx_v, scale_v, flags_v, bufs,
                   gs0, gs1, gs2, os0, os1, os2, psem):
    gsems = (gs0, gs1, gs2)
    osems = (os0, os1, os2)
    wid = lax.axis_index("s") * NC + lax.axis_index("c")
    base = wid * RPW

    pltpu.sync_copy(idx_hbm.at[pl.ds(base, RPW)], idx_v)
    sc_cp = pltpu.async_copy(
        scale_hbm.at[pl.ds(base * LANES, RPW * LANES)], scale_v, psem)
    fl_cp = pltpu.async_copy(
        flags_hbm.at[pl.ds(wid * NCHUNK, NCHUNK)], flags_v, psem)

    def gather(g):
        b = g % NBUF
        return pltpu.async_copy(
            x_hbm.at[idx_v.at[pl.ds(g * CH, CH)]], bufs.at[b], gsems[b])

    def scatter(g):
        b = g % NBUF
        return pltpu.async_copy(
            bufs.at[b], out_hbm.at[pl.ds(base + g * CH, CH)], osems[b])

    def compute(g):
        b = g % NBUF
        fvec = flags_v[pl.ds((g // LANES) * LANES, LANES)]
        flag = fvec[g % LANES]

        @pl.when(flag != 0)
        def _():
            def row_body(r, carry):
                svec = scale_v[pl.ds((g * CH + r) * LANES, LANES)]

                def col_body(j, carry2):
                    off = j * (LANES * UNROLL)
                    for u in range(UNROLL):
                        sl = pl.ds(off + u * LANES, LANES)
                        bufs[b, r, sl] = bufs[b, r, sl] * svec
                    return carry2

                return lax.fori_loop(0, D // (LANES * UNROLL), col_body, carry)

            lax.fori_loop(0, CH, row_body, 0)

    gc = {}
    oc = {}
    for g in range(min(NBUF, NCHUNK)):
        gc[g] = gather(g)
    sc_cp.wait()
    fl_cp.wait()
    for g in range(NCHUNK):
        gc[g].wait()
        compute(g)
        oc[g] = scatter(g)
        ng = g + NBUF - 1
        if NBUF <= ng < NCHUNK:
            oc[ng - NBUF].wait()
            gc[ng] = gather(ng)
    for g in range(max(0, NCHUNK - NBUF), NCHUNK):
        oc[g].wait()


def kernel(inputs):
    n = inputs.shape[0]
    key = jax.random.key(42)
    k_exec, k_flip, k_perm = jax.random.split(key, 3)
    execute = jax.random.uniform(k_exec, (), minval=0.0, maxval=1.0) < 0.1
    flip = jax.random.randint(k_flip, (n,), -1, 1).astype(jnp.float32)
    rotate_axis = jax.random.permutation(k_perm, n)
    idx = jnp.where(execute, rotate_axis,
                    jnp.arange(n, dtype=rotate_axis.dtype)).astype(jnp.int32)
    scale = jnp.where(execute, flip, jnp.ones((n,), jnp.float32))
    scale_exp = jnp.repeat(scale, LANES)
    flags = jnp.any(
        scale.reshape(n // CH, CH) != 1.0, axis=1).astype(jnp.int32)
    return _rotate_gather(inputs, idx, scale_exp, flags)
